# baseline (speedup 1.0000x reference)
import functools

import jax
import jax.numpy as jnp
from jax import lax
from jax.experimental import pallas as pl
from jax.experimental.pallas import tpu as pltpu
from jax.experimental.pallas import tpu_sc as plsc

_B, _M, _N, _L = 8, 256, 256, 128
_NSTRIPE = 4
_SROWS = _M // _NSTRIPE
_LANES = 16


def _roll_m1(a):
    return jnp.concatenate([a[:, 1:], a[:, :1]], axis=1)


def _roll_p1(a):
    return jnp.concatenate([a[:, -1:], a[:, :-1]], axis=1)


def _mean_der1(u, v):
    d1u = _roll_m1(u) - _roll_p1(u)
    d1v = _roll_m1(v) - _roll_p1(v)
    return jnp.mean(jnp.sqrt(d1u * d1u + d1v * d1v))


def _sqrt_nr(x):
    xi = plsc.bitcast(x, jnp.int32)
    y = plsc.bitcast(jnp.int32(0x5F3759DF) - (xi >> 1), jnp.float32)
    for _ in range(3):
        y = y * (1.5 - 0.5 * x * y * y)
    return x * y


def _sc_stamps(su, sv, tu, tv):
    mesh = plsc.VectorSubcoreMesh(core_axis_name="c", subcore_axis_name="s")

    @functools.partial(
        pl.kernel,
        out_type=(jax.ShapeDtypeStruct((_B, _M, _N), jnp.float32),
                  jax.ShapeDtypeStruct((_B, _M, _N), jnp.float32)),
        mesh=mesh,
        scratch_types=[pltpu.VMEM((_SROWS, _N), jnp.float32)] * 4 +
                      [pltpu.VMEM((_L + _LANES,), jnp.float32)] * 3,
        compiler_params=pltpu.CompilerParams(needs_layout_passes=False),
    )
    def k(su_h, sv_h, tu_h, tv_h, ge_h, gb_h, a1, a2, b1, b2,
          uref, vref, d2ref):
        wid = lax.axis_index("s") * 2 + lax.axis_index("c")
        b = wid // _NSTRIPE
        row_lo = (wid % _NSTRIPE) * _SROWS

        zeros16 = jnp.zeros((_LANES,), jnp.float32)

        def zbody(rr, _):
            for cc in range(_N // _LANES):
                sl = pl.ds(cc * _LANES, _LANES)
                a1[rr, sl] = zeros16
                a2[rr, sl] = zeros16
            return 0

        lax.fori_loop(0, _SROWS, zbody, 0)

        lane = lax.iota(jnp.int32, _LANES)

        def do_poly(uh, vh, img1, img2, zero_window):
            pltpu.sync_copy(uh.at[b], uref.at[pl.ds(0, _L)])
            pltpu.sync_copy(vh.at[b], vref.at[pl.ds(0, _L)])
            umin = None
            umax = None
            for ci in range(_L // _LANES):
                base = ci * _LANES
                idxm = lax.rem(lane + (base + _L - 1), _L)
                idxp = lax.rem(lane + (base + 1), _L)
                uc = uref[pl.ds(base, _LANES)]
                vc = vref[pl.ds(base, _LANES)]
                um = plsc.load_gather(uref, [idxm])
                up = plsc.load_gather(uref, [idxp])
                vm = plsc.load_gather(vref, [idxm])
                vp = plsc.load_gather(vref, [idxp])
                d2u = up + um - 2.0 * uc
                d2v = vp + vm - 2.0 * vc
                d2ref[pl.ds(base, _LANES)] = _sqrt_nr(d2u * d2u + d2v * d2v)
                cmin = lax.reduce_min(uc, (0,))
                cmax = lax.reduce_max(uc, (0,))
                umin = cmin if umin is None else jnp.minimum(umin, cmin)
                umax = cmax if umax is None else jnp.maximum(umax, cmax)

            umin = jnp.clip(umin, -1e9, 1e9)
            umax = jnp.clip(umax, -1e9, 1e9)
            w0 = jnp.clip(umin.astype(jnp.int32) - 3 - row_lo, 0, _SROWS)
            w1 = jnp.clip(umax.astype(jnp.int32) + 4 - row_lo, 0, _SROWS)

            if zero_window:
                def zb2(rr, _):
                    for cc in range(_N // _LANES):
                        sl = pl.ds(cc * _LANES, _LANES)
                        img1[rr, sl] = zeros16
                        img2[rr, sl] = zeros16
                    return 0

                lax.fori_loop(w0, w1, zb2, 0)

            def vbody(kk, _):
                idxk = lane * 0 + kk
                uk = plsc.load_gather(uref, [idxk])[0]
                lo_f = jnp.clip(uk - 2.0, -1e9, 1e9)
                hi_f = jnp.clip(uk + 2.0, -1e9, 1e9)
                ilo = lo_f.astype(jnp.int32)
                ilo = ilo + jnp.where(ilo.astype(jnp.float32) < lo_f, 1, 0)
                ihi = hi_f.astype(jnp.int32)
                ihi = ihi - jnp.where(ihi.astype(jnp.float32) > hi_f, 1, 0)
                r0 = jnp.maximum(ilo, row_lo)
                r1 = jnp.minimum(ihi, row_lo + _SROWS - 1)

                @pl.when(r0 <= r1)
                def _():
                    vk = plsc.load_gather(vref, [idxk])[0]
                    d2k = plsc.load_gather(d2ref, [idxk])[0]
                    vc_f = jnp.clip(vk, -1e9, 1e9)
                    iv = vc_f.astype(jnp.int32)
                    fv = iv - jnp.where(iv.astype(jnp.float32) > vc_f, 1, 0)
                    c0 = jnp.clip(fv - 2, 0, _N - _LANES)
                    cols = c0 + lane
                    xxv = cols.astype(jnp.float32)
                    dx = xxv - vk
                    dx2 = dx * dx

                    def rbody(r, _):
                        dy = r.astype(jnp.float32) - uk
                        m = dy * dy + dx2 <= 4.0
                        rows16 = lane * 0 + (r - row_lo)
                        cur1 = plsc.load_gather(img1, [rows16, cols])
                        plsc.store_scatter(img1, [rows16, cols],
                                           jnp.where(m, 1.0, cur1))
                        cur2 = plsc.load_gather(img2, [rows16, cols])
                        plsc.store_scatter(
                            img2, [rows16, cols],
                            jnp.maximum(cur2, jnp.where(m, d2k, 0.0)))
                        return 0

                    lax.fori_loop(r0, r1 + 1, rbody, 0)

                return 0

            lax.fori_loop(0, _L, vbody, 0)
            return w0, w1

        do_poly(su_h, sv_h, a1, a2, False)
        w0t, w1t = do_poly(tu_h, tv_h, b1, b2, True)

        def sbody(rr, _):
            for cc in range(_N // _LANES):
                sl = pl.ds(cc * _LANES, _LANES)
                a1[rr, sl] = a1[rr, sl] - b1[rr, sl]
                a2[rr, sl] = a2[rr, sl] - b2[rr, sl]
            return 0

        lax.fori_loop(w0t, w1t, sbody, 0)

        pltpu.sync_copy(a1, ge_h.at[b, pl.ds(row_lo, _SROWS)])
        pltpu.sync_copy(a2, gb_h.at[b, pl.ds(row_lo, _SROWS)])

    return k(su, sv, tu, tv)


_RBLK = 32


_ECHK = 16


def _rasterize(u, v, acc_scr):
    r, c = u, v
    r2, c2 = _roll_m1(u), _roll_m1(v)
    yy = jax.lax.broadcasted_iota(jnp.int32, (_M, 1), 0).astype(jnp.float32)
    cond = (r > yy) != (r2 > yy)
    denom = jnp.where(jnp.abs(r2 - r) < 1e-9, 1e-9, r2 - r)
    xint = c + (c2 - c) * (yy - r) / denom
    xint = jnp.where(cond, xint, -1.0)
    z = jnp.ceil(jnp.clip(xint, -1.0, 256.0)).astype(jnp.int32)
    rlo = jnp.minimum(r, r2)
    rhi = jnp.maximum(r, r2)
    xx = jax.lax.broadcasted_iota(jnp.int32, (_RBLK, _N), 1)
    acc_scr[:] = jnp.zeros((_M, _N), jnp.int32)
    for ck in range(_L // _ECHK):
        sl = slice(ck * _ECHK, (ck + 1) * _ECHK)
        cmin = jnp.min(rlo[:, sl])
        cmax = jnp.max(rhi[:, sl])
        for b in range(_M // _RBLK):
            blo = float(b * _RBLK)
            bhi = float((b + 1) * _RBLK)

            @pl.when((cmax > blo) & (cmin < bhi))
            def _():
                zb = z[b * _RBLK:(b + 1) * _RBLK, sl]
                acc = acc_scr[b * _RBLK:(b + 1) * _RBLK, :]
                for e in range(_ECHK):
                    acc = acc ^ (xx - zb[:, e:e + 1])
                acc_scr[b * _RBLK:(b + 1) * _RBLK, :] = acc

    return jnp.where(acc_scr[:] < 0, 1.0, 0.0)


def _loss_kernel(su_v, sv_v, tu_v, tv_v, tgt_ref,
                 ga_ref, gk_ref,
                 iou_ref, inter_ref, uni_ref, agt_ref, asn_ref, acc_scr):
    su = su_v[0]
    sv = sv_v[0]
    tu = tu_v[0]
    tv = tv_v[0]
    tgt = tgt_ref[0]

    mask = _rasterize(su, sv, acc_scr)
    s = tgt + mask
    isum = jnp.sum((s == 2.0).astype(jnp.int32))
    usum = jnp.sum((s >= 1.0).astype(jnp.int32))
    isum_f = isum.astype(jnp.float32)
    usum_f = usum.astype(jnp.float32)
    iou_ref[0, 0, 0] = isum_f / jnp.maximum(usum_f, 1.0)
    inter_ref[0, 0, 0] = isum_f / float(_M * _N)
    uni_ref[0, 0, 0] = usum_f / float(_M * _N)
    agt_ref[0, 0, 0] = jnp.sum((tgt > 0).astype(jnp.int32))
    asn_ref[0, 0, 0] = jnp.sum((mask > 0).astype(jnp.int32))

    gk_ref[0] = tgt - mask

    const = _mean_der1(su, sv) - _mean_der1(tu, tv)
    ga_ref[0] = jnp.zeros((_M, _N), jnp.float32) + const


def _run_tc(su3, sv3, tu3, tv3, tgt3, interpret=False):
    poly_vmem = pl.BlockSpec((1, 1, _L), lambda i: (i, 0, 0))
    img_spec = pl.BlockSpec((1, _M, _N), lambda i: (i, 0, 0))
    met_spec = pl.BlockSpec((1, 1, 1), lambda i: (i, 0, 0),
                            memory_space=pltpu.SMEM)
    f32 = jnp.float32
    i32 = jnp.int32
    out_shape = (
        jax.ShapeDtypeStruct((_B, _M, _N), f32),
        jax.ShapeDtypeStruct((_B, _M, _N), f32),
        jax.ShapeDtypeStruct((_B, 1, 1), f32),
        jax.ShapeDtypeStruct((_B, 1, 1), f32),
        jax.ShapeDtypeStruct((_B, 1, 1), f32),
        jax.ShapeDtypeStruct((_B, 1, 1), i32),
        jax.ShapeDtypeStruct((_B, 1, 1), i32),
    )
    return pl.pallas_call(
        _loss_kernel,
        grid=(_B,),
        in_specs=[poly_vmem, poly_vmem, poly_vmem, poly_vmem, img_spec],
        out_specs=(img_spec, img_spec,
                   met_spec, met_spec, met_spec, met_spec, met_spec),
        out_shape=out_shape,
        scratch_shapes=[pltpu.VMEM((_M, _N), jnp.int32)],
        compiler_params=pltpu.CompilerParams(
            dimension_semantics=("parallel",)),
        interpret=interpret,
    )(su3, sv3, tu3, tv3, tgt3)


def kernel(edges, alpha, beta, kappa, snakes, target_mask, target_snake,
           interpret=False):
    Bn, _, m, n = edges.shape
    su2 = snakes[:, :, 0]
    sv2 = snakes[:, :, 1]
    tu2 = target_snake[:, :, 0]
    tv2 = target_snake[:, :, 1]
    tgt3 = target_mask.reshape(Bn, m, n)
    ge, gb = _sc_stamps(su2, sv2, tu2, tv2)
    ga, gk, iou, inter, uni, agt, asn = _run_tc(
        su2.reshape(Bn, 1, _L), sv2.reshape(Bn, 1, _L),
        tu2.reshape(Bn, 1, _L), tv2.reshape(Bn, 1, _L),
        tgt3, interpret=interpret)
    return (ge[:, None], ga[:, None], gb[:, None], gk[:, None],
            iou.reshape(Bn), inter.reshape(Bn), uni.reshape(Bn),
            agt.reshape(Bn), asn.reshape(Bn))


# speedup vs baseline: 1.0133x; 1.0133x over previous
import functools

import jax
import jax.numpy as jnp
from jax import lax
from jax.experimental import pallas as pl
from jax.experimental.pallas import tpu as pltpu
from jax.experimental.pallas import tpu_sc as plsc

_B, _M, _N, _L = 8, 256, 256, 128
_NSTRIPE = 4
_SROWS = _M // _NSTRIPE
_LANES = 16


def _roll_m1(a):
    return jnp.concatenate([a[:, 1:], a[:, :1]], axis=1)


def _roll_p1(a):
    return jnp.concatenate([a[:, -1:], a[:, :-1]], axis=1)


def _mean_der1(u, v):
    d1u = _roll_m1(u) - _roll_p1(u)
    d1v = _roll_m1(v) - _roll_p1(v)
    return jnp.mean(jnp.sqrt(d1u * d1u + d1v * d1v))


def _sqrt_nr(x):
    xi = plsc.bitcast(x, jnp.int32)
    y = plsc.bitcast(jnp.int32(0x5F3759DF) - (xi >> 1), jnp.float32)
    for _ in range(3):
        y = y * (1.5 - 0.5 * x * y * y)
    return x * y


def _sc_stamps(su, sv, tu, tv):
    mesh = plsc.VectorSubcoreMesh(core_axis_name="c", subcore_axis_name="s")

    @functools.partial(
        pl.kernel,
        out_type=(jax.ShapeDtypeStruct((_B, _M, _N), jnp.float32),
                  jax.ShapeDtypeStruct((_B, _M, _N), jnp.float32)),
        mesh=mesh,
        scratch_types=[pltpu.VMEM((_SROWS, _N), jnp.float32)] * 4 +
                      [pltpu.VMEM((_L + _LANES,), jnp.float32)] * 3,
        compiler_params=pltpu.CompilerParams(needs_layout_passes=False),
    )
    def k(su_h, sv_h, tu_h, tv_h, ge_h, gb_h, a1, a2, b1, b2,
          uref, vref, d2ref):
        wid = lax.axis_index("s") * 2 + lax.axis_index("c")
        b = wid // _NSTRIPE
        row_lo = (wid % _NSTRIPE) * _SROWS

        zeros16 = jnp.zeros((_LANES,), jnp.float32)

        def zbody(rr, _):
            for cc in range(_N // _LANES):
                sl = pl.ds(cc * _LANES, _LANES)
                a1[rr, sl] = zeros16
                a2[rr, sl] = zeros16
            return 0

        lax.fori_loop(0, _SROWS, zbody, 0)

        lane = lax.iota(jnp.int32, _LANES)

        def do_poly(uh, vh, img1, img2, zero_window):
            pltpu.sync_copy(uh.at[b], uref.at[pl.ds(0, _L)])
            pltpu.sync_copy(vh.at[b], vref.at[pl.ds(0, _L)])
            umin = None
            umax = None
            for ci in range(_L // _LANES):
                base = ci * _LANES
                idxm = lax.rem(lane + (base + _L - 1), _L)
                idxp = lax.rem(lane + (base + 1), _L)
                uc = uref[pl.ds(base, _LANES)]
                vc = vref[pl.ds(base, _LANES)]
                um = plsc.load_gather(uref, [idxm])
                up = plsc.load_gather(uref, [idxp])
                vm = plsc.load_gather(vref, [idxm])
                vp = plsc.load_gather(vref, [idxp])
                d2u = up + um - 2.0 * uc
                d2v = vp + vm - 2.0 * vc
                d2ref[pl.ds(base, _LANES)] = _sqrt_nr(d2u * d2u + d2v * d2v)
                cmin = lax.reduce_min(uc, (0,))
                cmax = lax.reduce_max(uc, (0,))
                umin = cmin if umin is None else jnp.minimum(umin, cmin)
                umax = cmax if umax is None else jnp.maximum(umax, cmax)

            umin = jnp.clip(umin, -1e9, 1e9)
            umax = jnp.clip(umax, -1e9, 1e9)
            w0 = jnp.clip(umin.astype(jnp.int32) - 3 - row_lo, 0, _SROWS)
            w1 = jnp.clip(umax.astype(jnp.int32) + 4 - row_lo, 0, _SROWS)

            if zero_window:
                def zb2(rr, _):
                    for cc in range(_N // _LANES):
                        sl = pl.ds(cc * _LANES, _LANES)
                        img1[rr, sl] = zeros16
                        img2[rr, sl] = zeros16
                    return 0

                lax.fori_loop(w0, w1, zb2, 0)

            def vbody(kk, _):
                idxk = lane * 0 + kk
                uk = plsc.load_gather(uref, [idxk])[0]
                lo_f = jnp.clip(uk - 2.0, -1e9, 1e9)
                hi_f = jnp.clip(uk + 2.0, -1e9, 1e9)
                ilo = lo_f.astype(jnp.int32)
                ilo = ilo + jnp.where(ilo.astype(jnp.float32) < lo_f, 1, 0)
                ihi = hi_f.astype(jnp.int32)
                ihi = ihi - jnp.where(ihi.astype(jnp.float32) > hi_f, 1, 0)
                r0 = jnp.maximum(ilo, row_lo)
                r1 = jnp.minimum(ihi, row_lo + _SROWS - 1)

                @pl.when(r0 <= r1)
                def _():
                    vk = plsc.load_gather(vref, [idxk])[0]
                    d2k = plsc.load_gather(d2ref, [idxk])[0]
                    vc_f = jnp.clip(vk, -1e9, 1e9)
                    iv = vc_f.astype(jnp.int32)
                    fv = iv - jnp.where(iv.astype(jnp.float32) > vc_f, 1, 0)
                    c0 = jnp.clip(fv - 2, 0, _N - _LANES)
                    cols = c0 + lane
                    xxv = cols.astype(jnp.float32)
                    dx = xxv - vk
                    dx2 = dx * dx

                    def rbody(r, _):
                        dy = r.astype(jnp.float32) - uk
                        m = dy * dy + dx2 <= 4.0
                        rows16 = lane * 0 + (r - row_lo)
                        cur1 = plsc.load_gather(img1, [rows16, cols])
                        plsc.store_scatter(img1, [rows16, cols],
                                           jnp.where(m, 1.0, cur1))
                        cur2 = plsc.load_gather(img2, [rows16, cols])
                        plsc.store_scatter(
                            img2, [rows16, cols],
                            jnp.maximum(cur2, jnp.where(m, d2k, 0.0)))
                        return 0

                    lax.fori_loop(r0, r1 + 1, rbody, 0)

                return 0

            lax.fori_loop(0, _L, vbody, 0)
            return w0, w1

        do_poly(su_h, sv_h, a1, a2, False)
        w0t, w1t = do_poly(tu_h, tv_h, b1, b2, True)

        def sbody(rr, _):
            for cc in range(_N // _LANES):
                sl = pl.ds(cc * _LANES, _LANES)
                a1[rr, sl] = a1[rr, sl] - b1[rr, sl]
                a2[rr, sl] = a2[rr, sl] - b2[rr, sl]
            return 0

        lax.fori_loop(w0t, w1t, sbody, 0)

        pltpu.sync_copy(a1, ge_h.at[b, pl.ds(row_lo, _SROWS)])
        pltpu.sync_copy(a2, gb_h.at[b, pl.ds(row_lo, _SROWS)])

    return k(su, sv, tu, tv)


_RBLK = 32


_ECHK = 16


def _rasterize(u, v, acc_scr, z_scr):
    r, c = u, v
    r2, c2 = _roll_m1(u), _roll_m1(v)
    yy = jax.lax.broadcasted_iota(jnp.int32, (_M, 1), 0).astype(jnp.float32)
    cond = (r > yy) != (r2 > yy)
    denom = jnp.where(jnp.abs(r2 - r) < 1e-9, 1e-9, r2 - r)
    xint = c + (c2 - c) * (yy - r) / denom
    xint = jnp.where(cond, xint, -1.0)
    z = jnp.ceil(jnp.clip(xint, -1.0, 256.0)).astype(jnp.int32)
    rlo = jnp.minimum(r, r2)
    rhi = jnp.maximum(r, r2)
    xx = jax.lax.broadcasted_iota(jnp.int32, (_RBLK, _N), 1)
    acc_scr[:] = jnp.zeros((_M, _N), jnp.int32)
    z_scr[:] = z
    for ck in range(_L // _ECHK):
        sl = slice(ck * _ECHK, (ck + 1) * _ECHK)
        cmin = jnp.clip(jnp.min(rlo[:, sl]), -1e9, 1e9)
        cmax = jnp.clip(jnp.max(rhi[:, sl]), -1e9, 1e9)
        it = cmin.astype(jnp.int32)
        ymin = it + jnp.where(it.astype(jnp.float32) < cmin, 1, 0)
        it2 = cmax.astype(jnp.int32)
        ymax = it2 + jnp.where(it2.astype(jnp.float32) < cmax, 1, 0) - 1
        blo = jnp.clip(ymin, 0, _M) >> 5
        bub = (jnp.clip(ymax, -1, _M - 1) >> 5) + 1

        def bbody(bb, _, _sl=sl):
            r0 = bb * _RBLK
            zb = z_scr[pl.ds(r0, _RBLK), _sl]
            acc = acc_scr[pl.ds(r0, _RBLK), :]
            for e in range(_ECHK):
                acc = acc ^ (xx - zb[:, e:e + 1])
            acc_scr[pl.ds(r0, _RBLK), :] = acc
            return 0

        lax.fori_loop(blo, bub, bbody, 0)

    return jnp.where(acc_scr[:] < 0, 1.0, 0.0)


def _loss_kernel(su_v, sv_v, tu_v, tv_v, tgt_ref,
                 ga_ref, gk_ref,
                 iou_ref, inter_ref, uni_ref, agt_ref, asn_ref,
                 acc_scr, z_scr):
    su = su_v[0]
    sv = sv_v[0]
    tu = tu_v[0]
    tv = tv_v[0]
    tgt = tgt_ref[0]

    mask = _rasterize(su, sv, acc_scr, z_scr)
    s = tgt + mask
    isum = jnp.sum((s == 2.0).astype(jnp.int32))
    usum = jnp.sum((s >= 1.0).astype(jnp.int32))
    isum_f = isum.astype(jnp.float32)
    usum_f = usum.astype(jnp.float32)
    iou_ref[0, 0, 0] = isum_f / jnp.maximum(usum_f, 1.0)
    inter_ref[0, 0, 0] = isum_f / float(_M * _N)
    uni_ref[0, 0, 0] = usum_f / float(_M * _N)
    agt_ref[0, 0, 0] = jnp.sum((tgt > 0).astype(jnp.int32))
    asn_ref[0, 0, 0] = jnp.sum((mask > 0).astype(jnp.int32))

    gk_ref[0] = tgt - mask

    const = _mean_der1(su, sv) - _mean_der1(tu, tv)
    ga_ref[0] = jnp.zeros((_M, _N), jnp.float32) + const


def _run_tc(su3, sv3, tu3, tv3, tgt3, interpret=False):
    poly_vmem = pl.BlockSpec((1, 1, _L), lambda i: (i, 0, 0))
    img_spec = pl.BlockSpec((1, _M, _N), lambda i: (i, 0, 0))
    met_spec = pl.BlockSpec((1, 1, 1), lambda i: (i, 0, 0),
                            memory_space=pltpu.SMEM)
    f32 = jnp.float32
    i32 = jnp.int32
    out_shape = (
        jax.ShapeDtypeStruct((_B, _M, _N), f32),
        jax.ShapeDtypeStruct((_B, _M, _N), f32),
        jax.ShapeDtypeStruct((_B, 1, 1), f32),
        jax.ShapeDtypeStruct((_B, 1, 1), f32),
        jax.ShapeDtypeStruct((_B, 1, 1), f32),
        jax.ShapeDtypeStruct((_B, 1, 1), i32),
        jax.ShapeDtypeStruct((_B, 1, 1), i32),
    )
    return pl.pallas_call(
        _loss_kernel,
        grid=(_B,),
        in_specs=[poly_vmem, poly_vmem, poly_vmem, poly_vmem, img_spec],
        out_specs=(img_spec, img_spec,
                   met_spec, met_spec, met_spec, met_spec, met_spec),
        out_shape=out_shape,
        scratch_shapes=[pltpu.VMEM((_M, _N), jnp.int32),
                        pltpu.VMEM((_M, _L), jnp.int32)],
        compiler_params=pltpu.CompilerParams(
            dimension_semantics=("parallel",)),
        interpret=interpret,
    )(su3, sv3, tu3, tv3, tgt3)


def kernel(edges, alpha, beta, kappa, snakes, target_mask, target_snake,
           interpret=False):
    Bn, _, m, n = edges.shape
    su2 = snakes[:, :, 0]
    sv2 = snakes[:, :, 1]
    tu2 = target_snake[:, :, 0]
    tv2 = target_snake[:, :, 1]
    tgt3 = target_mask.reshape(Bn, m, n)
    ge, gb = _sc_stamps(su2, sv2, tu2, tv2)
    ga, gk, iou, inter, uni, agt, asn = _run_tc(
        su2.reshape(Bn, 1, _L), sv2.reshape(Bn, 1, _L),
        tu2.reshape(Bn, 1, _L), tv2.reshape(Bn, 1, _L),
        tgt3, interpret=interpret)
    return (ge[:, None], ga[:, None], gb[:, None], gk[:, None],
            iou.reshape(Bn), inter.reshape(Bn), uni.reshape(Bn),
            agt.reshape(Bn), asn.reshape(Bn))


# speedup vs baseline: 1.0153x; 1.0020x over previous
import functools

import jax
import jax.numpy as jnp
from jax import lax
from jax.experimental import pallas as pl
from jax.experimental.pallas import tpu as pltpu
from jax.experimental.pallas import tpu_sc as plsc

_B, _M, _N, _L = 8, 256, 256, 128
_NSTRIPE = 4
_SROWS = _M // _NSTRIPE
_LANES = 16


def _roll_m1(a):
    return jnp.concatenate([a[:, 1:], a[:, :1]], axis=1)


def _roll_p1(a):
    return jnp.concatenate([a[:, -1:], a[:, :-1]], axis=1)


def _mean_der1(u, v):
    d1u = _roll_m1(u) - _roll_p1(u)
    d1v = _roll_m1(v) - _roll_p1(v)
    return jnp.mean(jnp.sqrt(d1u * d1u + d1v * d1v))


def _sqrt_nr(x):
    xi = plsc.bitcast(x, jnp.int32)
    y = plsc.bitcast(jnp.int32(0x5F3759DF) - (xi >> 1), jnp.float32)
    for _ in range(3):
        y = y * (1.5 - 0.5 * x * y * y)
    return x * y


def _sc_stamps(su, sv, tu, tv):
    mesh = plsc.VectorSubcoreMesh(core_axis_name="c", subcore_axis_name="s")

    @functools.partial(
        pl.kernel,
        out_type=(jax.ShapeDtypeStruct((_B, _M, _N), jnp.float32),
                  jax.ShapeDtypeStruct((_B, _M, _N), jnp.float32)),
        mesh=mesh,
        scratch_types=[pltpu.VMEM((_SROWS, _N), jnp.float32)] * 4 +
                      [pltpu.VMEM((_L + _LANES,), jnp.float32)] * 3,
        compiler_params=pltpu.CompilerParams(needs_layout_passes=False),
    )
    def k(su_h, sv_h, tu_h, tv_h, ge_h, gb_h, a1, a2, b1, b2,
          uref, vref, d2ref):
        wid = lax.axis_index("s") * 2 + lax.axis_index("c")
        b = wid // _NSTRIPE
        row_lo = (wid % _NSTRIPE) * _SROWS

        zeros16 = jnp.zeros((_LANES,), jnp.float32)

        def zbody(rr, _):
            for cc in range(_N // _LANES):
                sl = pl.ds(cc * _LANES, _LANES)
                a1[rr, sl] = zeros16
                a2[rr, sl] = zeros16
            return 0

        lax.fori_loop(0, _SROWS, zbody, 0)

        lane = lax.iota(jnp.int32, _LANES)

        def do_poly(uh, vh, img1, img2, zero_window):
            pltpu.sync_copy(uh.at[b], uref.at[pl.ds(0, _L)])
            pltpu.sync_copy(vh.at[b], vref.at[pl.ds(0, _L)])
            umin = None
            umax = None
            for ci in range(_L // _LANES):
                base = ci * _LANES
                idxm = lax.rem(lane + (base + _L - 1), _L)
                idxp = lax.rem(lane + (base + 1), _L)
                uc = uref[pl.ds(base, _LANES)]
                vc = vref[pl.ds(base, _LANES)]
                um = plsc.load_gather(uref, [idxm])
                up = plsc.load_gather(uref, [idxp])
                vm = plsc.load_gather(vref, [idxm])
                vp = plsc.load_gather(vref, [idxp])
                d2u = up + um - 2.0 * uc
                d2v = vp + vm - 2.0 * vc
                d2ref[pl.ds(base, _LANES)] = _sqrt_nr(d2u * d2u + d2v * d2v)
                cmin = lax.reduce_min(uc, (0,))
                cmax = lax.reduce_max(uc, (0,))
                umin = cmin if umin is None else jnp.minimum(umin, cmin)
                umax = cmax if umax is None else jnp.maximum(umax, cmax)

            umin = jnp.clip(umin, -1e9, 1e9)
            umax = jnp.clip(umax, -1e9, 1e9)
            w0 = jnp.clip(umin.astype(jnp.int32) - 3 - row_lo, 0, _SROWS)
            w1 = jnp.clip(umax.astype(jnp.int32) + 4 - row_lo, 0, _SROWS)

            if zero_window:
                def zb2(rr, _):
                    for cc in range(_N // _LANES):
                        sl = pl.ds(cc * _LANES, _LANES)
                        img1[rr, sl] = zeros16
                        img2[rr, sl] = zeros16
                    return 0

                lax.fori_loop(w0, w1, zb2, 0)

            def vbody(kk, _):
                idxk = lane * 0 + kk
                uk = plsc.load_gather(uref, [idxk])[0]
                lo_f = jnp.clip(uk - 2.0, -1e9, 1e9)
                hi_f = jnp.clip(uk + 2.0, -1e9, 1e9)
                ilo = lo_f.astype(jnp.int32)
                ilo = ilo + jnp.where(ilo.astype(jnp.float32) < lo_f, 1, 0)
                ihi = hi_f.astype(jnp.int32)
                ihi = ihi - jnp.where(ihi.astype(jnp.float32) > hi_f, 1, 0)
                r0 = jnp.maximum(ilo, row_lo)
                r1 = jnp.minimum(ihi, row_lo + _SROWS - 1)

                @pl.when(r0 <= r1)
                def _():
                    vk = plsc.load_gather(vref, [idxk])[0]
                    d2k = plsc.load_gather(d2ref, [idxk])[0]
                    vc_f = jnp.clip(vk, -1e9, 1e9)
                    iv = vc_f.astype(jnp.int32)
                    fv = iv - jnp.where(iv.astype(jnp.float32) > vc_f, 1, 0)
                    c0 = jnp.clip(fv - 2, 0, _N - _LANES)
                    cols = c0 + lane
                    xxv = cols.astype(jnp.float32)
                    dx = xxv - vk
                    dx2 = dx * dx

                    def rbody(r, _):
                        dy = r.astype(jnp.float32) - uk
                        m = dy * dy + dx2 <= 4.0
                        rows16 = lane * 0 + (r - row_lo)
                        cur1 = plsc.load_gather(img1, [rows16, cols])
                        plsc.store_scatter(img1, [rows16, cols],
                                           jnp.where(m, 1.0, cur1))
                        cur2 = plsc.load_gather(img2, [rows16, cols])
                        plsc.store_scatter(
                            img2, [rows16, cols],
                            jnp.maximum(cur2, jnp.where(m, d2k, 0.0)))
                        return 0

                    lax.fori_loop(r0, r1 + 1, rbody, 0)

                return 0

            lax.fori_loop(0, _L, vbody, 0)
            return w0, w1

        do_poly(su_h, sv_h, a1, a2, False)
        w0t, w1t = do_poly(tu_h, tv_h, b1, b2, True)

        def sbody(rr, _):
            for cc in range(_N // _LANES):
                sl = pl.ds(cc * _LANES, _LANES)
                a1[rr, sl] = a1[rr, sl] - b1[rr, sl]
                a2[rr, sl] = a2[rr, sl] - b2[rr, sl]
            return 0

        lax.fori_loop(w0t, w1t, sbody, 0)

        pltpu.sync_copy(a1, ge_h.at[b, pl.ds(row_lo, _SROWS)])
        pltpu.sync_copy(a2, gb_h.at[b, pl.ds(row_lo, _SROWS)])

    return k(su, sv, tu, tv)


_RBLK = 32


_ECHK = 16


def _rasterize(u, v, acc_scr, z_scr):
    r, c = u, v
    r2, c2 = _roll_m1(u), _roll_m1(v)
    yy = jax.lax.broadcasted_iota(jnp.int32, (_M, 1), 0).astype(jnp.float32)
    cond = (r > yy) != (r2 > yy)
    denom = jnp.where(jnp.abs(r2 - r) < 1e-9, 1e-9, r2 - r)
    xint = c + (c2 - c) * (yy - r) / denom
    xint = jnp.where(cond, xint, -1.0)
    z = jnp.ceil(jnp.clip(xint, -1.0, 256.0)).astype(jnp.int32)
    rlo = jnp.minimum(r, r2)
    rhi = jnp.maximum(r, r2)
    xx = jax.lax.broadcasted_iota(jnp.int32, (_RBLK, _N), 1)
    acc_scr[:] = jnp.zeros((_M, _N), jnp.int32)
    z_scr[:] = z
    for ck in range(_L // _ECHK):
        sl = slice(ck * _ECHK, (ck + 1) * _ECHK)
        cmin = jnp.clip(jnp.min(rlo[:, sl]), -1e9, 1e9)
        cmax = jnp.clip(jnp.max(rhi[:, sl]), -1e9, 1e9)
        it = cmin.astype(jnp.int32)
        ymin = it + jnp.where(it.astype(jnp.float32) < cmin, 1, 0)
        it2 = cmax.astype(jnp.int32)
        ymax = it2 + jnp.where(it2.astype(jnp.float32) < cmax, 1, 0) - 1
        blo = jnp.clip(ymin, 0, _M) >> 5
        bub = (jnp.clip(ymax, -1, _M - 1) >> 5) + 1

        def bbody(bb, _, _sl=sl):
            r0 = bb * _RBLK
            zb = z_scr[pl.ds(r0, _RBLK), _sl]
            acc = acc_scr[pl.ds(r0, _RBLK), :]
            for e in range(_ECHK):
                acc = acc ^ (xx - zb[:, e:e + 1])
            acc_scr[pl.ds(r0, _RBLK), :] = acc
            return 0

        lax.fori_loop(blo, bub, bbody, 0)

    return jnp.where(acc_scr[:] < 0, 1.0, 0.0)


def _loss_kernel(su_v, sv_v, tu_v, tv_v, tgt_ref,
                 ga_ref, gk_ref,
                 iou_ref, inter_ref, uni_ref, agt_ref, asn_ref,
                 acc_scr, z_scr):
    for b in range(_B):
        su = su_v[b]
        sv = sv_v[b]
        tu = tu_v[b]
        tv = tv_v[b]
        tgt = tgt_ref[b]

        mask = _rasterize(su, sv, acc_scr, z_scr)
        s = tgt + mask
        isum = jnp.sum((s == 2.0).astype(jnp.int32))
        usum = jnp.sum((s >= 1.0).astype(jnp.int32))
        isum_f = isum.astype(jnp.float32)
        usum_f = usum.astype(jnp.float32)
        iou_ref[b, 0, 0] = isum_f / jnp.maximum(usum_f, 1.0)
        inter_ref[b, 0, 0] = isum_f / float(_M * _N)
        uni_ref[b, 0, 0] = usum_f / float(_M * _N)
        agt_ref[b, 0, 0] = jnp.sum((tgt > 0).astype(jnp.int32))
        asn_ref[b, 0, 0] = jnp.sum((mask > 0).astype(jnp.int32))

        gk_ref[b] = tgt - mask

        const = _mean_der1(su, sv) - _mean_der1(tu, tv)
        ga_ref[b] = jnp.zeros((_M, _N), jnp.float32) + const


def _run_tc(su3, sv3, tu3, tv3, tgt3, interpret=False):
    f32 = jnp.float32
    i32 = jnp.int32
    met_spec = pl.BlockSpec(memory_space=pltpu.SMEM)
    out_shape = (
        jax.ShapeDtypeStruct((_B, _M, _N), f32),
        jax.ShapeDtypeStruct((_B, _M, _N), f32),
        jax.ShapeDtypeStruct((_B, 1, 1), f32),
        jax.ShapeDtypeStruct((_B, 1, 1), f32),
        jax.ShapeDtypeStruct((_B, 1, 1), f32),
        jax.ShapeDtypeStruct((_B, 1, 1), i32),
        jax.ShapeDtypeStruct((_B, 1, 1), i32),
    )
    return pl.pallas_call(
        _loss_kernel,
        out_specs=(pl.BlockSpec(), pl.BlockSpec(),
                   met_spec, met_spec, met_spec, met_spec, met_spec),
        out_shape=out_shape,
        scratch_shapes=[pltpu.VMEM((_M, _N), jnp.int32),
                        pltpu.VMEM((_M, _L), jnp.int32)],
        interpret=interpret,
    )(su3, sv3, tu3, tv3, tgt3)


def kernel(edges, alpha, beta, kappa, snakes, target_mask, target_snake,
           interpret=False):
    Bn, _, m, n = edges.shape
    su2 = snakes[:, :, 0]
    sv2 = snakes[:, :, 1]
    tu2 = target_snake[:, :, 0]
    tv2 = target_snake[:, :, 1]
    tgt3 = target_mask.reshape(Bn, m, n)
    ge, gb = _sc_stamps(su2, sv2, tu2, tv2)
    ga, gk, iou, inter, uni, agt, asn = _run_tc(
        su2.reshape(Bn, 1, _L), sv2.reshape(Bn, 1, _L),
        tu2.reshape(Bn, 1, _L), tv2.reshape(Bn, 1, _L),
        tgt3, interpret=interpret)
    return (ge[:, None], ga[:, None], gb[:, None], gk[:, None],
            iou.reshape(Bn), inter.reshape(Bn), uni.reshape(Bn),
            agt.reshape(Bn), asn.reshape(Bn))
